# Initial kernel scaffold; baseline (speedup 1.0000x reference)
import jax
import jax.numpy as jnp
from jax.experimental import pallas as pl


def kernel(query, layer_input, edges, nodes, edge_count, rel_embed, Wt, bt, Wh):
    raise NotImplementedError("write your pallas kernel here")


# speedup vs baseline: 11.1776x; 11.1776x over previous
import functools

import jax
import jax.numpy as jnp
from jax import lax
from jax.experimental import pallas as pl
from jax.experimental.pallas import tpu as pltpu
from jax.experimental.pallas import tpu_sc as plsc

N = 10000
E = 320000
D = 128
B = 32
R = 402

NC = 2
NS = 16
NW = NC * NS
EPW = E // NW
K = 80
NBLK = EPW // K
NP = 10240
ROWS_PER_TILE = NP // NS
ZROWS = 128


def _relation_body(q_ref, re_ref, wt_ref, bt_ref, out_ref):
    q = q_ref[...]
    re_ = re_ref[...]
    wt = wt_ref[...]
    bt = bt_ref[...]
    dn = (((1,), (1,)), ((), ()))
    qw = lax.dot_general(q, wt[:, :D], dn, preferred_element_type=jnp.float32)
    rw = lax.dot_general(re_, wt[:, D:], dn, preferred_element_type=jnp.float32)
    rel3 = qw[:, None, :] + rw[None, :, :] + bt[None, :, :]
    out_ref[...] = rel3.reshape(B * R, D)


def _relation_table(query, rel_embed, Wt, bt):
    return pl.pallas_call(
        _relation_body,
        out_shape=jax.ShapeDtypeStruct((B * R, D), jnp.float32),
    )(query, rel_embed, Wt, bt.reshape(1, D))


def _sc_body(layer_hbm, rel_hbm, eflat_hbm, out_hbm,
             qbuf, rbuf, subidx, flatidx, objidx, in_rows, rel_rows, msg, zbuf,
             agg_sh, sem_a, sem_b):
    cid = lax.axis_index("c")
    sid = lax.axis_index("s")
    wid = sid * NC + cid
    ebase = wid * EPW

    def zrow(i, _):
        for c in range(D // 16):
            zbuf[i, pl.ds(c * 16, 16)] = jnp.zeros((16,), jnp.float32)
        return _
    lax.fori_loop(0, ZROWS, zrow, None)
    for t in range(ROWS_PER_TILE // ZROWS):
        pltpu.sync_copy(zbuf,
                        agg_sh.at[pl.ds(sid * ROWS_PER_TILE + t * ZROWS, ZROWS), :])
    plsc.subcore_barrier()

    def block(j, _):
        base = ebase + j * K
        pltpu.sync_copy(eflat_hbm.at[pl.ds(0 * E + base, K)], qbuf)
        pltpu.sync_copy(eflat_hbm.at[pl.ds(2 * E + base, K)], rbuf)
        pltpu.sync_copy(eflat_hbm.at[pl.ds(4 * E + base, K)], subidx)
        pltpu.sync_copy(eflat_hbm.at[pl.ds(5 * E + base, K)], objidx)
        for t in range(K // 16):
            sl = pl.ds(t * 16, 16)
            flatidx[sl] = qbuf[sl] * R + rbuf[sl]
        cp_a = pltpu.async_copy(layer_hbm.at[subidx], in_rows, sem_a)
        cp_b = pltpu.async_copy(rel_hbm.at[flatidx], rel_rows, sem_b)
        cp_a.wait()
        cp_b.wait()
        def row(r, _):
            for c in range(D // 16):
                sl = pl.ds(c * 16, 16)
                msg[r, sl] = in_rows[r, sl] * rel_rows[r, sl]
            return _
        lax.fori_loop(0, K, row, None)
        pltpu.sync_copy(msg, agg_sh.at[objidx], add=True)
        return _
    lax.fori_loop(0, NBLK, block, None)

    plsc.subcore_barrier()
    pltpu.sync_copy(agg_sh.at[pl.ds(sid * ROWS_PER_TILE, ROWS_PER_TILE), :],
                    out_hbm.at[cid, pl.ds(sid * ROWS_PER_TILE, ROWS_PER_TILE), :])


def _sc_aggregate(layer_input, rel_flat, eflat):
    mesh = plsc.VectorSubcoreMesh(core_axis_name="c", subcore_axis_name="s",
                                  num_cores=NC, num_subcores=NS)
    kern = pl.kernel(
        _sc_body,
        out_type=jax.ShapeDtypeStruct((NC, NP, D), jnp.float32),
        mesh=mesh,
        scratch_types=[
            pltpu.VMEM((K,), jnp.int32),
            pltpu.VMEM((K,), jnp.int32),
            pltpu.VMEM((K,), jnp.int32),
            pltpu.VMEM((K,), jnp.int32),
            pltpu.VMEM((K,), jnp.int32),
            pltpu.VMEM((K, D), jnp.float32),
            pltpu.VMEM((K, D), jnp.float32),
            pltpu.VMEM((K, D), jnp.float32),
            pltpu.VMEM((ZROWS, D), jnp.float32),
            pltpu.VMEM_SHARED((NP, D), jnp.float32),
            pltpu.SemaphoreType.DMA,
            pltpu.SemaphoreType.DMA,
        ],
    )
    return kern(layer_input, rel_flat, eflat)


def _output_body(a0_ref, a1_ref, cnt_ref, wh_ref, out_ref):
    s = a0_ref[...] + a1_ref[...]
    cnt = jnp.clip(cnt_ref[...], 1.0, None)
    s = s / cnt
    h = lax.dot_general(s, wh_ref[...], (((1,), (1,)), ((), ())),
                        preferred_element_type=jnp.float32)
    m = jnp.mean(h, axis=-1, keepdims=True)
    v = jnp.mean((h - m) ** 2, axis=-1, keepdims=True)
    out_ref[...] = jnp.maximum((h - m) / jnp.sqrt(v + 1e-5), 0.0)


_OUT_BLK = 2000


def _output(agg0, agg1, edge_count, Wh):
    nblk = N // _OUT_BLK
    return pl.pallas_call(
        _output_body,
        grid=(nblk,),
        in_specs=[
            pl.BlockSpec((_OUT_BLK, D), lambda j: (j, 0)),
            pl.BlockSpec((_OUT_BLK, D), lambda j: (j, 0)),
            pl.BlockSpec((_OUT_BLK, 1), lambda j: (j, 0)),
            pl.BlockSpec((D, D), lambda j: (0, 0)),
        ],
        out_specs=pl.BlockSpec((_OUT_BLK, D), lambda j: (j, 0)),
        out_shape=jax.ShapeDtypeStruct((N, D), jnp.float32),
    )(agg0, agg1, edge_count, Wh)


def kernel(query, layer_input, edges, nodes, edge_count, rel_embed, Wt, bt, Wh):
    rel_flat = _relation_table(query, rel_embed, Wt, bt)
    aggs = _sc_aggregate(layer_input, rel_flat, edges.T.reshape(-1))
    return _output(aggs[0], aggs[1], edge_count, Wh)


# speedup vs baseline: 25.4968x; 2.2811x over previous
import functools

import jax
import jax.numpy as jnp
from jax import lax
from jax.experimental import pallas as pl
from jax.experimental.pallas import tpu as pltpu
from jax.experimental.pallas import tpu_sc as plsc

N = 10000
E = 320000
D = 128
B = 32
R = 402

NC = 2
NS = 16
NW = NC * NS
EPW = E // NW
K = 80
NBLK = EPW // K
NBUF = 2
NP = 10240
ROWS_PER_TILE = NP // NS
ZROWS = 16


def _relation_body(q_ref, re_ref, wt_ref, bt_ref, et_ref, out_ref, flat_ref):
    q = q_ref[...]
    re_ = re_ref[...]
    wt = wt_ref[...]
    bt = bt_ref[...]
    dn = (((1,), (1,)), ((), ()))
    qw = lax.dot_general(q, wt[:, :D], dn, preferred_element_type=jnp.float32)
    rw = lax.dot_general(re_, wt[:, D:], dn, preferred_element_type=jnp.float32)
    rel3 = qw[:, None, :] + rw[None, :, :] + bt[None, :, :]
    out_ref[...] = rel3.reshape(B * R, D)
    flat_ref[...] = et_ref[0:1, :] * R + et_ref[2:3, :]


def _relation_table(query, rel_embed, Wt, bt, edgesT):
    return pl.pallas_call(
        _relation_body,
        out_shape=(jax.ShapeDtypeStruct((B * R, D), jnp.float32),
                   jax.ShapeDtypeStruct((1, E), jnp.int32)),
    )(query, rel_embed, Wt, bt.reshape(1, D), edgesT)


KH = K // 2


def _sc_body(layer_hbm, rel_hbm, sub_hbm, flat_hbm, obj_hbm, out_hbm,
             subk, flatk, objk, in_rows, rel_rows, zbuf,
             agg_sh, isem, osem, gsem_a, gsem_b, ssem):
    cid = lax.axis_index("c")
    sid = lax.axis_index("s")
    wid = sid * NC + cid
    ebase = wid * EPW

    def zrow(i, _):
        for c in range(D // 16):
            zbuf[i, pl.ds(c * 16, 16)] = jnp.zeros((16,), jnp.float32)
        return _
    lax.fori_loop(0, ZROWS, zrow, None)
    for t in range(ROWS_PER_TILE // ZROWS):
        pltpu.sync_copy(zbuf,
                        agg_sh.at[pl.ds(sid * ROWS_PER_TILE + t * ZROWS, ZROWS), :])
    plsc.subcore_barrier()

    def fire_sf_idx(g, b):
        off = pl.multiple_of(ebase + g * K, 8)
        pltpu.async_copy(sub_hbm.at[pl.ds(off, K)], subk[b], isem[b])
        pltpu.async_copy(flat_hbm.at[pl.ds(off, K)], flatk[b], isem[b])

    def wait_sf_idx(b):
        pltpu.make_async_copy(sub_hbm.at[pl.ds(0, K)], subk[b], isem[b]).wait()
        pltpu.make_async_copy(flat_hbm.at[pl.ds(0, K)], flatk[b], isem[b]).wait()

    def fire_obj_idx(g, b):
        off = pl.multiple_of(ebase + g * K, 8)
        pltpu.async_copy(obj_hbm.at[pl.ds(off, KH)], objk[2 * b], osem[b])
        pltpu.async_copy(obj_hbm.at[pl.ds(off + KH, KH)],
                         objk[2 * b + 1], osem[b])

    def wait_obj_idx(b):
        pltpu.make_async_copy(obj_hbm.at[pl.ds(0, KH)], objk[0], osem[b]).wait()
        pltpu.make_async_copy(obj_hbm.at[pl.ds(0, KH)], objk[0], osem[b]).wait()

    def fire_gathers(b):
        pltpu.async_copy(layer_hbm.at[subk[b]], in_rows[b], gsem_a[b])
        pltpu.async_copy(rel_hbm.at[flatk[b]], rel_rows[b], gsem_b[b])

    def wait_gathers(b):
        pltpu.make_async_copy(
            layer_hbm.at[subk[b]], in_rows[b], gsem_a[b]).wait()
        pltpu.make_async_copy(
            rel_hbm.at[flatk[b]], rel_rows[b], gsem_b[b]).wait()

    def compute_half(b, h):
        def row(r, _):
            for c in range(D // 16):
                sl = pl.ds(c * 16, 16)
                in_rows[b][r, sl] = in_rows[b][r, sl] * rel_rows[b][r, sl]
            return _
        lax.fori_loop(h * KH, h * KH + KH, row, None)

    def fire_scatter_half(b, h):
        pltpu.async_copy(in_rows[b].at[pl.ds(h * KH, KH), :],
                         agg_sh.at[objk[2 * b + h]], ssem[b], add=True)

    def wait_scatters(b):
        for h in range(2):
            pltpu.make_async_copy(in_rows[b].at[pl.ds(0, KH), :],
                                  agg_sh.at[objk[2 * b]], ssem[b]).wait()

    def handle(g, b, last):
        wait_gathers(b)
        if not last:
            @pl.when(g + NBUF < NBLK)
            def _():
                fire_sf_idx(g + NBUF, b)
        compute_half(b, 0)
        wait_obj_idx(b)
        fire_scatter_half(b, 0)
        compute_half(b, 1)
        fire_scatter_half(b, 1)
        wait_scatters(b)
        if not last:
            @pl.when(g + NBUF < NBLK)
            def _():
                fire_obj_idx(g + NBUF, b)
                wait_sf_idx(b)
                fire_gathers(b)

    for b in range(NBUF):
        fire_sf_idx(b, b)
        fire_obj_idx(b, b)
    for b in range(NBUF):
        wait_sf_idx(b)
        fire_gathers(b)

    def body(i, _):
        for b in range(NBUF):
            handle(i * NBUF + b, b, last=False)
        return _
    lax.fori_loop(0, NBLK // NBUF, body, None)
    for g in range(NBLK - NBLK % NBUF, NBLK):
        handle(g, g % NBUF, last=True)

    plsc.subcore_barrier()
    pltpu.sync_copy(agg_sh.at[pl.ds(sid * ROWS_PER_TILE, ROWS_PER_TILE), :],
                    out_hbm.at[cid, pl.ds(sid * ROWS_PER_TILE, ROWS_PER_TILE), :])


def _sc_aggregate(layer_input, rel_flat, subcol, flatcol, objcol):
    mesh = plsc.VectorSubcoreMesh(core_axis_name="c", subcore_axis_name="s",
                                  num_cores=NC, num_subcores=NS)
    kern = pl.kernel(
        _sc_body,
        out_type=jax.ShapeDtypeStruct((NC, NP, D), jnp.float32),
        mesh=mesh,
        scratch_types=[
            [pltpu.VMEM((K,), jnp.int32)] * NBUF,
            [pltpu.VMEM((K,), jnp.int32)] * NBUF,
            [pltpu.VMEM((KH,), jnp.int32)] * (NBUF * 2),
            [pltpu.VMEM((K, D), jnp.float32)] * NBUF,
            [pltpu.VMEM((K, D), jnp.float32)] * NBUF,
            pltpu.VMEM((ZROWS, D), jnp.float32),
            pltpu.VMEM_SHARED((NP, D), jnp.float32),
            [pltpu.SemaphoreType.DMA] * NBUF,
            [pltpu.SemaphoreType.DMA] * NBUF,
            [pltpu.SemaphoreType.DMA] * NBUF,
            [pltpu.SemaphoreType.DMA] * NBUF,
            [pltpu.SemaphoreType.DMA] * NBUF,
        ],
    )
    return kern(layer_input, rel_flat, subcol, flatcol, objcol)


def _output_body(a0_ref, a1_ref, cnt_ref, wh_ref, out_ref):
    s = a0_ref[...] + a1_ref[...]
    cnt = jnp.clip(cnt_ref[...], 1.0, None)
    s = s / cnt
    h = lax.dot_general(s, wh_ref[...], (((1,), (1,)), ((), ())),
                        preferred_element_type=jnp.float32)
    m = jnp.mean(h, axis=-1, keepdims=True)
    v = jnp.mean((h - m) ** 2, axis=-1, keepdims=True)
    out_ref[...] = jnp.maximum((h - m) / jnp.sqrt(v + 1e-5), 0.0)


_OUT_BLK = 2000


def _output(agg0, agg1, edge_count, Wh):
    nblk = N // _OUT_BLK
    return pl.pallas_call(
        _output_body,
        grid=(nblk,),
        in_specs=[
            pl.BlockSpec((_OUT_BLK, D), lambda j: (j, 0)),
            pl.BlockSpec((_OUT_BLK, D), lambda j: (j, 0)),
            pl.BlockSpec((_OUT_BLK, 1), lambda j: (j, 0)),
            pl.BlockSpec((D, D), lambda j: (0, 0)),
        ],
        out_specs=pl.BlockSpec((_OUT_BLK, D), lambda j: (j, 0)),
        out_shape=jax.ShapeDtypeStruct((N, D), jnp.float32),
    )(agg0, agg1, edge_count, Wh)


def kernel(query, layer_input, edges, nodes, edge_count, rel_embed, Wt, bt, Wh):
    edgesT = edges.T
    rel_flat, flat = _relation_table(query, rel_embed, Wt, bt, edgesT)
    aggs = _sc_aggregate(layer_input, rel_flat, edgesT[4], flat.reshape(-1),
                         edgesT[5])
    return _output(aggs[0], aggs[1], edge_count, Wh)


# speedup vs baseline: 26.2142x; 1.0281x over previous
import functools

import jax
import jax.numpy as jnp
from jax import lax
from jax.experimental import pallas as pl
from jax.experimental.pallas import tpu as pltpu
from jax.experimental.pallas import tpu_sc as plsc

N = 10000
E = 320000
D = 128
B = 32
R = 402

NC = 2
NS = 16
NW = NC * NS
EPW = E // NW
K = 80
NBLK = EPW // K
NBUF = 2
NP = 10240
ROWS_PER_TILE = NP // NS
ZROWS = 16


def _relation_body(q_ref, re_ref, wt_ref, bt_ref, et_ref, out_ref, flat_ref):
    q = q_ref[...]
    re_ = re_ref[...]
    wt = wt_ref[...]
    bt = bt_ref[...]
    dn = (((1,), (1,)), ((), ()))
    qw = lax.dot_general(q, wt[:, :D], dn, preferred_element_type=jnp.float32)
    rw = lax.dot_general(re_, wt[:, D:], dn, preferred_element_type=jnp.float32)
    rel3 = qw[:, None, :] + rw[None, :, :] + bt[None, :, :]
    out_ref[...] = rel3.reshape(B * R, D)
    flat_ref[...] = et_ref[0:1, :] * R + et_ref[2:3, :]


def _relation_table(query, rel_embed, Wt, bt, edgesT):
    return pl.pallas_call(
        _relation_body,
        out_shape=(jax.ShapeDtypeStruct((B * R, D), jnp.float32),
                   jax.ShapeDtypeStruct((1, E), jnp.int32)),
    )(query, rel_embed, Wt, bt.reshape(1, D), edgesT)


KH = K // 2


def _sc_body(layer_hbm, rel_hbm, sub_hbm, flat_hbm, obj_hbm, out_hbm,
             subk, flatk, objk, in_rows, rel_rows, zbuf,
             agg_sh, isem, osem, gsem_a, gsem_b, ssem):
    cid = lax.axis_index("c")
    sid = lax.axis_index("s")
    wid = sid * NC + cid
    ebase = wid * EPW

    def zrow(i, _):
        for c in range(D // 16):
            zbuf[i, pl.ds(c * 16, 16)] = jnp.zeros((16,), jnp.float32)
        return _
    lax.fori_loop(0, ZROWS, zrow, None)
    for t in range(ROWS_PER_TILE // ZROWS):
        pltpu.sync_copy(zbuf,
                        agg_sh.at[pl.ds(sid * ROWS_PER_TILE + t * ZROWS, ZROWS), :])
    plsc.subcore_barrier()

    def fire_sf_idx(g, b):
        off = pl.multiple_of(ebase + g * K, 8)
        pltpu.async_copy(sub_hbm.at[pl.ds(off, K)], subk[b], isem[b])
        pltpu.async_copy(flat_hbm.at[pl.ds(off, K)], flatk[b], isem[b])

    def wait_sf_idx(b):
        pltpu.make_async_copy(sub_hbm.at[pl.ds(0, K)], subk[b], isem[b]).wait()
        pltpu.make_async_copy(flat_hbm.at[pl.ds(0, K)], flatk[b], isem[b]).wait()

    def fire_obj_idx(g, b):
        off = pl.multiple_of(ebase + g * K, 8)
        pltpu.async_copy(obj_hbm.at[pl.ds(off, KH)], objk[2 * b], osem[b])
        pltpu.async_copy(obj_hbm.at[pl.ds(off + KH, KH)],
                         objk[2 * b + 1], osem[b])

    def wait_obj_idx(b):
        pltpu.make_async_copy(obj_hbm.at[pl.ds(0, KH)], objk[0], osem[b]).wait()
        pltpu.make_async_copy(obj_hbm.at[pl.ds(0, KH)], objk[0], osem[b]).wait()

    def fire_gathers(b):
        pltpu.async_copy(layer_hbm.at[subk[b]], in_rows[b], gsem_a[b])
        pltpu.async_copy(rel_hbm.at[flatk[b]], rel_rows[b], gsem_b[b])

    def wait_gathers(b):
        pltpu.make_async_copy(
            layer_hbm.at[subk[b]], in_rows[b], gsem_a[b]).wait()
        pltpu.make_async_copy(
            rel_hbm.at[flatk[b]], rel_rows[b], gsem_b[b]).wait()

    def compute_half(b, h):
        def row(r, _):
            for c in range(D // 16):
                sl = pl.ds(c * 16, 16)
                in_rows[b][r, sl] = in_rows[b][r, sl] * rel_rows[b][r, sl]
            return _
        lax.fori_loop(h * KH, h * KH + KH, row, None)

    def fire_scatter_half(b, h):
        pltpu.async_copy(in_rows[b].at[pl.ds(h * KH, KH), :],
                         agg_sh.at[objk[2 * b + h]], ssem[b], add=True)

    def wait_scatters(b):
        for h in range(2):
            pltpu.make_async_copy(in_rows[b].at[pl.ds(0, KH), :],
                                  agg_sh.at[objk[2 * b]], ssem[b]).wait()

    def handle(g, b, last):
        wait_gathers(b)
        if not last:
            @pl.when(g + NBUF < NBLK)
            def _():
                fire_sf_idx(g + NBUF, b)
        compute_half(b, 0)
        wait_obj_idx(b)
        fire_scatter_half(b, 0)
        compute_half(b, 1)
        fire_scatter_half(b, 1)
        wait_scatters(b)
        if not last:
            @pl.when(g + NBUF < NBLK)
            def _():
                fire_obj_idx(g + NBUF, b)
                wait_sf_idx(b)
                fire_gathers(b)

    for b in range(NBUF):
        fire_sf_idx(b, b)
        fire_obj_idx(b, b)
    for b in range(NBUF):
        wait_sf_idx(b)
        fire_gathers(b)

    def body(i, _):
        for b in range(NBUF):
            handle(i * NBUF + b, b, last=False)
        return _
    lax.fori_loop(0, NBLK // NBUF, body, None)
    for g in range(NBLK - NBLK % NBUF, NBLK):
        handle(g, g % NBUF, last=True)

    plsc.subcore_barrier()
    pltpu.sync_copy(agg_sh.at[pl.ds(sid * ROWS_PER_TILE, ROWS_PER_TILE), :],
                    out_hbm.at[cid, pl.ds(sid * ROWS_PER_TILE, ROWS_PER_TILE), :])


def _sc_aggregate(layer_input, rel_flat, subcol, flatcol, objcol):
    mesh = plsc.VectorSubcoreMesh(core_axis_name="c", subcore_axis_name="s",
                                  num_cores=NC, num_subcores=NS)
    kern = pl.kernel(
        _sc_body,
        out_type=jax.ShapeDtypeStruct((NC, NP, D), jnp.float32),
        mesh=mesh,
        scratch_types=[
            [pltpu.VMEM((K,), jnp.int32)] * NBUF,
            [pltpu.VMEM((K,), jnp.int32)] * NBUF,
            [pltpu.VMEM((KH,), jnp.int32)] * (NBUF * 2),
            [pltpu.VMEM((K, D), jnp.float32)] * NBUF,
            [pltpu.VMEM((K, D), jnp.float32)] * NBUF,
            pltpu.VMEM((ZROWS, D), jnp.float32),
            pltpu.VMEM_SHARED((NP, D), jnp.float32),
            [pltpu.SemaphoreType.DMA] * NBUF,
            [pltpu.SemaphoreType.DMA] * NBUF,
            [pltpu.SemaphoreType.DMA] * NBUF,
            [pltpu.SemaphoreType.DMA] * NBUF,
            [pltpu.SemaphoreType.DMA] * NBUF,
        ],
    )
    return kern(layer_input, rel_flat, subcol, flatcol, objcol)


def _output_body(agg_ref, cnt_ref, wh_ref, out_ref):
    s = agg_ref[0] + agg_ref[1]
    cnt = jnp.clip(cnt_ref[...], 1.0, None)
    s = s / cnt
    h = lax.dot_general(s, wh_ref[...], (((1,), (1,)), ((), ())),
                        preferred_element_type=jnp.float32)
    m = jnp.mean(h, axis=-1, keepdims=True)
    v = jnp.mean((h - m) ** 2, axis=-1, keepdims=True)
    out_ref[...] = jnp.maximum((h - m) / jnp.sqrt(v + 1e-5), 0.0)


_OUT_BLK = 2000


def _output(aggs, edge_count, Wh):
    nblk = N // _OUT_BLK
    return pl.pallas_call(
        _output_body,
        grid=(nblk,),
        in_specs=[
            pl.BlockSpec((NC, _OUT_BLK, D), lambda j: (0, j, 0)),
            pl.BlockSpec((_OUT_BLK, 1), lambda j: (j, 0)),
            pl.BlockSpec((D, D), lambda j: (0, 0)),
        ],
        out_specs=pl.BlockSpec((_OUT_BLK, D), lambda j: (j, 0)),
        out_shape=jax.ShapeDtypeStruct((N, D), jnp.float32),
    )(aggs, edge_count, Wh)


def kernel(query, layer_input, edges, nodes, edge_count, rel_embed, Wt, bt, Wh):
    edgesT = edges.T
    rel_flat, flat = _relation_table(query, rel_embed, Wt, bt, edgesT)
    aggs = _sc_aggregate(layer_input, rel_flat, edgesT[4], flat.reshape(-1),
                         edgesT[5])
    return _output(aggs, edge_count, Wh)
